# baseline (speedup 1.0000x reference)
import jax
import jax.numpy as jnp
from jax import lax
from jax.experimental import pallas as pl
from jax.experimental.pallas import tpu as pltpu
from jax.experimental.pallas import tpu_sc as plsc

N_NODES = 50000
IN_FEATS = 512
NUM_GRAPHS = 256
NW = 32
PER_W = 1568
N_PAD = NW * PER_W
CHUNK = 192
L = 16


def _set_idx(idx_buf, cur):
    lanes = lax.iota(jnp.int32, L)
    plsc.store_scatter(idx_buf, [jnp.zeros((L,), jnp.int32)],
                       jnp.full((L,), cur, jnp.int32), mask=lanes == 0)


def _zero_acc(acc_buf):
    z = jnp.zeros((L,), jnp.float32)
    for j in range(IN_FEATS // L):
        acc_buf[0, pl.ds(j * L, L)] = z


def _dot16(row_flat, wsplat, b_s, g16):
    idx0 = (g16 + lax.iota(jnp.int32, L)) * IN_FEATS
    ones = jnp.ones((L,), jnp.int32)
    zero = jnp.zeros((L,), jnp.float32)

    def jgbody(jg, carry):
        a0, a1, a2, a3, idx = carry
        accs = [a0, a1, a2, a3]
        for l in range(L):
            col = plsc.load_gather(row_flat, [idx])
            wv = wsplat[jg * L + l, :]
            accs[l % 4] = accs[l % 4] + col * wv
            idx = idx + ones
        return accs[0], accs[1], accs[2], accs[3], idx

    a0, a1, a2, a3, _ = lax.fori_loop(
        0, IN_FEATS // L, jgbody, (zero, zero, zero, zero, idx0))
    return (a0 + a1) + (a2 + a3) + b_s


def _process_chunk(pos, n_groups, cur_seg, feats_flat,
                   row_flat, ids_buf, aw_buf, w_buf, acc_buf, idx_buf,
                   wsplat, b_s, hg_acc, start):
    nrows = n_groups * L
    loc = pos - start
    pltpu.sync_copy(feats_flat.at[pl.ds(pos * IN_FEATS, nrows * IN_FEATS)],
                    row_flat.at[pl.ds(0, nrows * IN_FEATS)])
    for g in range(n_groups):
        aw16 = _dot16(row_flat, wsplat, b_s, g * L)
        w16 = 1.0 / (1.0 + jnp.exp(-aw16))
        aw_buf[pl.ds(loc + g * L, L)] = aw16
        w_buf[pl.ds(loc + g * L, L)] = w16

    def rbody(r, cur):
        s_r = ids_buf[pl.ds(loc + r, L)][0]
        w_r = w_buf[pl.ds(loc + r, L)][0]

        @pl.when(s_r != cur)
        def _():
            _set_idx(idx_buf, cur)
            pltpu.sync_copy(acc_buf, hg_acc.at[idx_buf], add=True)
            _zero_acc(acc_buf)

        for j in range(IN_FEATS // L):
            acc_buf[0, pl.ds(j * L, L)] += (
                w_r * row_flat[pl.ds(r * IN_FEATS + j * L, L)])
        return s_r

    return lax.fori_loop(0, nrows, rbody, cur_seg)


def _sc_body(feats_flat, ids, wsplat_hbm, b_hbm, zeros_hbm, aw_out, hg_part,
             row_flat, ids_buf, aw_buf, w_buf, acc_buf, idx_buf,
             wsplat_vmem, b_vmem, hg_acc):
    cid = lax.axis_index("c")
    sid = lax.axis_index("s")
    wid = cid * 16 + sid

    pltpu.sync_copy(wsplat_hbm, wsplat_vmem)
    pltpu.sync_copy(b_hbm, b_vmem)
    b_s = b_vmem[...][0]

    @pl.when(sid == 0)
    def _():
        pltpu.sync_copy(zeros_hbm, hg_acc)

    plsc.subcore_barrier()
    _zero_acc(acc_buf)

    start = wid * PER_W
    count = jnp.minimum(PER_W, N_NODES - start)
    nfull = count // CHUNK
    ntail = (count - nfull * CHUNK) // L

    pltpu.sync_copy(ids.at[pl.ds(start, PER_W)], ids_buf.at[pl.ds(0, PER_W)])

    args = (feats_flat, row_flat, ids_buf, aw_buf, w_buf, acc_buf,
            idx_buf, wsplat_vmem, b_s, hg_acc, start)

    def full_body(k, cur):
        return _process_chunk(start + k * CHUNK, CHUNK // L, cur, *args)

    cur_seg = lax.fori_loop(0, nfull, full_body, jnp.int32(NUM_GRAPHS))

    def tail_body(k, cur):
        return _process_chunk(start + nfull * CHUNK + k * L, 1, cur, *args)

    cur_seg = lax.fori_loop(0, ntail, tail_body, cur_seg)

    _set_idx(idx_buf, cur_seg)
    pltpu.sync_copy(acc_buf, hg_acc.at[idx_buf], add=True)

    pltpu.sync_copy(aw_buf.at[pl.ds(0, PER_W)], aw_out.at[pl.ds(start, PER_W)])

    plsc.subcore_barrier()
    off = cid * NUM_GRAPHS + sid * L
    pltpu.sync_copy(hg_acc.at[pl.ds(sid * L, L)], hg_part.at[pl.ds(off, L)])


@jax.jit
def _sc_call(feats_flat, ids_pad, wsplat, b16, zeros):
    mesh = plsc.VectorSubcoreMesh(core_axis_name="c", subcore_axis_name="s",
                                  num_cores=2, num_subcores=16)
    return pl.kernel(
        _sc_body,
        out_type=[
            jax.ShapeDtypeStruct((N_PAD,), jnp.float32),
            jax.ShapeDtypeStruct((2 * NUM_GRAPHS, IN_FEATS), jnp.float32),
        ],
        mesh=mesh,
        compiler_params=pltpu.CompilerParams(use_tc_tiling_on_sc=False,
                                             needs_layout_passes=False),
        scratch_types=[
            pltpu.VMEM((CHUNK * IN_FEATS,), jnp.float32),
            pltpu.VMEM((PER_W + L,), jnp.int32),
            pltpu.VMEM((PER_W + L,), jnp.float32),
            pltpu.VMEM((PER_W + L,), jnp.float32),
            pltpu.VMEM((1, IN_FEATS), jnp.float32),
            pltpu.VMEM((1,), jnp.int32),
            pltpu.VMEM((IN_FEATS, L), jnp.float32),
            pltpu.VMEM((L,), jnp.float32),
            pltpu.VMEM_SHARED((NUM_GRAPHS + 1, IN_FEATS), jnp.float32),
        ],
    )(feats_flat, ids_pad, wsplat, b16, zeros)


def _merge_body(p_ref, out_ref):
    out_ref[...] = p_ref[0] + p_ref[1]


def kernel(feats, segment_ids, W, b):
    ids_pad = jnp.pad(segment_ids.astype(jnp.int32), (0, N_PAD - N_NODES),
                      constant_values=NUM_GRAPHS - 1)
    wsplat = jnp.tile(W.reshape(IN_FEATS, 1).astype(jnp.float32), (1, L))
    b16 = jnp.concatenate([b.astype(jnp.float32),
                           jnp.zeros((L - 1,), jnp.float32)])
    zeros = jnp.zeros((NUM_GRAPHS + 1, IN_FEATS), jnp.float32)
    aw_flat, hg_part = _sc_call(feats.reshape(N_NODES * IN_FEATS), ids_pad,
                                wsplat, b16, zeros)
    hg = pl.pallas_call(
        _merge_body,
        out_shape=jax.ShapeDtypeStruct((NUM_GRAPHS, IN_FEATS), jnp.float32),
    )(hg_part.reshape(2, NUM_GRAPHS, IN_FEATS))
    return (hg, aw_flat[:N_NODES].reshape(N_NODES, 1))


# speedup vs baseline: 1.5568x; 1.5568x over previous
import jax
import jax.numpy as jnp
from jax import lax
from jax.experimental import pallas as pl
from jax.experimental.pallas import tpu as pltpu
from jax.experimental.pallas import tpu_sc as plsc

N_NODES = 50000
IN_FEATS = 512
NUM_GRAPHS = 256
NW = 32
PER_W = 1568
N_PAD = NW * PER_W
CHUNK = 192
L = 16
BLOCK = 2000
NUM_BLOCKS = N_NODES // BLOCK


def _matvec_body(f_ref, w_ref, b_ref, aw_ref, sig_ref):
    aw = jax.lax.dot_general(
        f_ref[...], w_ref[...], (((1,), (0,)), ((), ())),
        preferred_element_type=jnp.float32,
    ) + b_ref[0, 0]
    aw_ref[...] = aw
    sig_ref[...] = jax.nn.sigmoid(aw)


def _tc_matvec(feats, W, b2):
    return pl.pallas_call(
        _matvec_body,
        grid=(NUM_BLOCKS,),
        in_specs=[
            pl.BlockSpec((BLOCK, IN_FEATS), lambda i: (i, 0)),
            pl.BlockSpec((IN_FEATS, 1), lambda i: (0, 0)),
            pl.BlockSpec((1, 1), lambda i: (0, 0)),
        ],
        out_specs=[
            pl.BlockSpec((BLOCK, 1), lambda i: (i, 0)),
            pl.BlockSpec((BLOCK, 1), lambda i: (i, 0)),
        ],
        out_shape=[
            jax.ShapeDtypeStruct((N_NODES, 1), jnp.float32),
            jax.ShapeDtypeStruct((N_NODES, 1), jnp.float32),
        ],
    )(feats, W, b2)


def _set_idx(idx_buf, cur):
    lanes = lax.iota(jnp.int32, L)
    plsc.store_scatter(idx_buf, [jnp.zeros((L,), jnp.int32)],
                       jnp.full((L,), cur, jnp.int32), mask=lanes == 0)


def _zero_acc(acc_buf):
    z = jnp.zeros((L,), jnp.float32)
    for j in range(IN_FEATS // L):
        acc_buf[0, pl.ds(j * L, L)] = z


def _process_chunk(pos, n_groups, cur_seg, feats_flat,
                   row_flat, ids_buf, w_buf, acc_buf, idx_buf, hg_acc, start):
    nrows = n_groups * L
    loc = pos - start
    pltpu.sync_copy(feats_flat.at[pl.ds(pos * IN_FEATS, nrows * IN_FEATS)],
                    row_flat.at[pl.ds(0, nrows * IN_FEATS)])

    def rbody(r, cur):
        s_r = ids_buf[pl.ds(loc + r, L)][0]
        w_r = w_buf[pl.ds(loc + r, L)][0]

        @pl.when(s_r != cur)
        def _():
            _set_idx(idx_buf, cur)
            pltpu.sync_copy(acc_buf, hg_acc.at[idx_buf], add=True)
            _zero_acc(acc_buf)

        for j in range(IN_FEATS // L):
            acc_buf[0, pl.ds(j * L, L)] += (
                w_r * row_flat[pl.ds(r * IN_FEATS + j * L, L)])
        return s_r

    return lax.fori_loop(0, nrows, rbody, cur_seg)


def _sc_body(feats_flat, ids, wvals, zeros_hbm, hg_part,
             row_flat, ids_buf, w_buf, acc_buf, idx_buf, hg_acc):
    cid = lax.axis_index("c")
    sid = lax.axis_index("s")
    wid = cid * 16 + sid

    @pl.when(sid == 0)
    def _():
        pltpu.sync_copy(zeros_hbm, hg_acc)

    plsc.subcore_barrier()
    _zero_acc(acc_buf)

    start = wid * PER_W
    count = jnp.minimum(PER_W, N_NODES - start)
    nfull = count // CHUNK
    ntail = (count - nfull * CHUNK) // L

    pltpu.sync_copy(ids.at[pl.ds(start, PER_W)], ids_buf.at[pl.ds(0, PER_W)])
    pltpu.sync_copy(wvals.at[pl.ds(start, PER_W)], w_buf.at[pl.ds(0, PER_W)])

    args = (feats_flat, row_flat, ids_buf, w_buf, acc_buf, idx_buf, hg_acc,
            start)

    def full_body(k, cur):
        return _process_chunk(start + k * CHUNK, CHUNK // L, cur, *args)

    cur_seg = lax.fori_loop(0, nfull, full_body, jnp.int32(NUM_GRAPHS))

    def tail_body(k, cur):
        return _process_chunk(start + nfull * CHUNK + k * L, 1, cur, *args)

    cur_seg = lax.fori_loop(0, ntail, tail_body, cur_seg)

    _set_idx(idx_buf, cur_seg)
    pltpu.sync_copy(acc_buf, hg_acc.at[idx_buf], add=True)

    plsc.subcore_barrier()
    off = cid * NUM_GRAPHS + sid * L
    pltpu.sync_copy(hg_acc.at[pl.ds(sid * L, L)], hg_part.at[pl.ds(off, L)])


@jax.jit
def _sc_call(feats_flat, ids_pad, w_pad, zeros):
    mesh = plsc.VectorSubcoreMesh(core_axis_name="c", subcore_axis_name="s",
                                  num_cores=2, num_subcores=16)
    return pl.kernel(
        _sc_body,
        out_type=jax.ShapeDtypeStruct((2 * NUM_GRAPHS, IN_FEATS),
                                      jnp.float32),
        mesh=mesh,
        compiler_params=pltpu.CompilerParams(use_tc_tiling_on_sc=False,
                                             needs_layout_passes=False),
        scratch_types=[
            pltpu.VMEM((CHUNK * IN_FEATS,), jnp.float32),
            pltpu.VMEM((PER_W + L,), jnp.int32),
            pltpu.VMEM((PER_W + L,), jnp.float32),
            pltpu.VMEM((1, IN_FEATS), jnp.float32),
            pltpu.VMEM((1,), jnp.int32),
            pltpu.VMEM_SHARED((NUM_GRAPHS + 1, IN_FEATS), jnp.float32),
        ],
    )(feats_flat, ids_pad, w_pad, zeros)


def _merge_body(p_ref, out_ref):
    out_ref[...] = p_ref[0] + p_ref[1]


def kernel(feats, segment_ids, W, b):
    b2 = b.reshape(1, 1).astype(jnp.float32)
    aw, wv = _tc_matvec(feats, W, b2)
    ids_pad = jnp.pad(segment_ids.astype(jnp.int32), (0, N_PAD - N_NODES),
                      constant_values=NUM_GRAPHS - 1)
    w_pad = jnp.pad(wv.reshape(N_NODES), (0, N_PAD - N_NODES))
    zeros = jnp.zeros((NUM_GRAPHS + 1, IN_FEATS), jnp.float32)
    hg_part = _sc_call(feats.reshape(N_NODES * IN_FEATS), ids_pad, w_pad,
                       zeros)
    hg = pl.pallas_call(
        _merge_body,
        out_shape=jax.ShapeDtypeStruct((NUM_GRAPHS, IN_FEATS), jnp.float32),
    )(hg_part.reshape(2, NUM_GRAPHS, IN_FEATS))
    return (hg, aw)


# speedup vs baseline: 2.9090x; 1.8686x over previous
import jax
import jax.numpy as jnp
from jax import lax
from jax.experimental import pallas as pl
from jax.experimental.pallas import tpu as pltpu
from jax.experimental.pallas import tpu_sc as plsc

N_NODES = 50000
IN_FEATS = 512
NUM_GRAPHS = 256
NW = 32
PER_W = 1568
N_PAD = NW * PER_W
CHUNK = 192
L = 16
BLOCK = 2000
NUM_BLOCKS = N_NODES // BLOCK


def _matvec_body(f_ref, w_ref, b_ref, aw_ref, sig_ref):
    aw = jax.lax.dot_general(
        f_ref[...], w_ref[...], (((1,), (0,)), ((), ())),
        preferred_element_type=jnp.float32,
    ) + b_ref[0, 0]
    aw_ref[...] = aw
    sig_ref[...] = jax.nn.sigmoid(aw)


def _tc_matvec(feats, W, b2):
    return pl.pallas_call(
        _matvec_body,
        grid=(NUM_BLOCKS,),
        in_specs=[
            pl.BlockSpec((BLOCK, IN_FEATS), lambda i: (i, 0)),
            pl.BlockSpec((IN_FEATS, 1), lambda i: (0, 0)),
            pl.BlockSpec((1, 1), lambda i: (0, 0)),
        ],
        out_specs=[
            pl.BlockSpec((BLOCK, 1), lambda i: (i, 0)),
            pl.BlockSpec((BLOCK, 1), lambda i: (i, 0)),
        ],
        out_shape=[
            jax.ShapeDtypeStruct((N_NODES, 1), jnp.float32),
            jax.ShapeDtypeStruct((N_NODES, 1), jnp.float32),
        ],
    )(feats, W, b2)


def _set_idx(idx_buf, cur):
    lanes = lax.iota(jnp.int32, L)
    plsc.store_scatter(idx_buf, [jnp.zeros((L,), jnp.int32)],
                       jnp.full((L,), cur, jnp.int32), mask=lanes == 0)


NJ = IN_FEATS // L


def _flush(accs, acc_buf, idx_buf, hg_acc, cur):
    for j in range(NJ):
        acc_buf[0, pl.ds(j * L, L)] = accs[j]
    _set_idx(idx_buf, cur)
    pltpu.sync_copy(acc_buf, hg_acc.at[idx_buf], add=True)


def _process_chunk(pos, n_groups, carry, feats_flat,
                   row_flat, ids_buf, w_buf, acc_buf, idx_buf, hg_acc, start):
    nrows = n_groups * L
    loc = pos - start
    pltpu.sync_copy(feats_flat.at[pl.ds(pos * IN_FEATS, nrows * IN_FEATS)],
                    row_flat.at[pl.ds(0, nrows * IN_FEATS)])

    def rbody(r, carry):
        cur = carry[0]
        accs = carry[1:]
        s_r = ids_buf[pl.ds(loc + r, L)][0]
        w_r = w_buf[pl.ds(loc + r, L)][0]
        changed = s_r != cur

        @pl.when(changed)
        def _():
            _flush(accs, acc_buf, idx_buf, hg_acc, cur)

        new_accs = tuple(
            jnp.where(changed,
                      w_r * row_flat[pl.ds(r * IN_FEATS + j * L, L)],
                      accs[j] + w_r * row_flat[pl.ds(r * IN_FEATS + j * L, L)])
            for j in range(NJ))
        return (s_r,) + new_accs

    return lax.fori_loop(0, nrows, rbody, carry)


def _sc_body(feats_flat, ids, wvals, zeros_hbm, hg_part,
             row_flat, ids_buf, w_buf, acc_buf, idx_buf, hg_acc):
    cid = lax.axis_index("c")
    sid = lax.axis_index("s")
    wid = cid * 16 + sid

    @pl.when(sid == 0)
    def _():
        pltpu.sync_copy(zeros_hbm, hg_acc)

    plsc.subcore_barrier()

    start = wid * PER_W
    count = jnp.minimum(PER_W, N_NODES - start)
    nfull = count // CHUNK
    ntail = (count - nfull * CHUNK) // L

    pltpu.sync_copy(ids.at[pl.ds(start, PER_W)], ids_buf.at[pl.ds(0, PER_W)])
    pltpu.sync_copy(wvals.at[pl.ds(start, PER_W)], w_buf.at[pl.ds(0, PER_W)])

    args = (feats_flat, row_flat, ids_buf, w_buf, acc_buf, idx_buf, hg_acc,
            start)

    zero = jnp.zeros((L,), jnp.float32)
    carry0 = (jnp.int32(NUM_GRAPHS),) + (zero,) * NJ

    def full_body(k, carry):
        return _process_chunk(start + k * CHUNK, CHUNK // L, carry, *args)

    carry = lax.fori_loop(0, nfull, full_body, carry0)

    def tail_body(k, carry):
        return _process_chunk(start + nfull * CHUNK + k * L, 1, carry, *args)

    carry = lax.fori_loop(0, ntail, tail_body, carry)

    _flush(carry[1:], acc_buf, idx_buf, hg_acc, carry[0])

    plsc.subcore_barrier()
    off = cid * NUM_GRAPHS + sid * L
    pltpu.sync_copy(hg_acc.at[pl.ds(sid * L, L)], hg_part.at[pl.ds(off, L)])


@jax.jit
def _sc_call(feats_flat, ids_pad, w_pad, zeros):
    mesh = plsc.VectorSubcoreMesh(core_axis_name="c", subcore_axis_name="s",
                                  num_cores=2, num_subcores=16)
    return pl.kernel(
        _sc_body,
        out_type=jax.ShapeDtypeStruct((2 * NUM_GRAPHS, IN_FEATS),
                                      jnp.float32),
        mesh=mesh,
        compiler_params=pltpu.CompilerParams(use_tc_tiling_on_sc=False,
                                             needs_layout_passes=False),
        scratch_types=[
            pltpu.VMEM((CHUNK * IN_FEATS,), jnp.float32),
            pltpu.VMEM((PER_W + L,), jnp.int32),
            pltpu.VMEM((PER_W + L,), jnp.float32),
            pltpu.VMEM((1, IN_FEATS), jnp.float32),
            pltpu.VMEM((1,), jnp.int32),
            pltpu.VMEM_SHARED((NUM_GRAPHS + 1, IN_FEATS), jnp.float32),
        ],
    )(feats_flat, ids_pad, w_pad, zeros)


def _merge_body(p_ref, out_ref):
    out_ref[...] = p_ref[0] + p_ref[1]


def kernel(feats, segment_ids, W, b):
    b2 = b.reshape(1, 1).astype(jnp.float32)
    aw, wv = _tc_matvec(feats, W, b2)
    ids_pad = jnp.pad(segment_ids.astype(jnp.int32), (0, N_PAD - N_NODES),
                      constant_values=NUM_GRAPHS - 1)
    w_pad = jnp.pad(wv.reshape(N_NODES), (0, N_PAD - N_NODES))
    zeros = jnp.zeros((NUM_GRAPHS + 1, IN_FEATS), jnp.float32)
    hg_part = _sc_call(feats.reshape(N_NODES * IN_FEATS), ids_pad, w_pad,
                       zeros)
    hg = pl.pallas_call(
        _merge_body,
        out_shape=jax.ShapeDtypeStruct((NUM_GRAPHS, IN_FEATS), jnp.float32),
    )(hg_part.reshape(2, NUM_GRAPHS, IN_FEATS))
    return (hg, aw)


# speedup vs baseline: 3.1415x; 1.0799x over previous
import jax
import jax.numpy as jnp
from jax import lax
from jax.experimental import pallas as pl
from jax.experimental.pallas import tpu as pltpu
from jax.experimental.pallas import tpu_sc as plsc

N_NODES = 50000
IN_FEATS = 512
NUM_GRAPHS = 256
NW = 32
PER_W = 1568
N_PAD = NW * PER_W
CHUNK = 192
L = 16
BLOCK = 2000
NUM_BLOCKS = N_NODES // BLOCK


def _matvec_body(f_ref, w_ref, b_ref, aw_ref, sig_ref):
    aw = jax.lax.dot_general(
        f_ref[...], w_ref[...], (((1,), (0,)), ((), ())),
        preferred_element_type=jnp.float32,
    ) + b_ref[0, 0]
    aw_ref[...] = aw
    sig_ref[...] = jax.nn.sigmoid(aw)


def _tc_matvec(feats, W, b2):
    return pl.pallas_call(
        _matvec_body,
        grid=(NUM_BLOCKS,),
        in_specs=[
            pl.BlockSpec((BLOCK, IN_FEATS), lambda i: (i, 0)),
            pl.BlockSpec((IN_FEATS, 1), lambda i: (0, 0)),
            pl.BlockSpec((1, 1), lambda i: (0, 0)),
        ],
        out_specs=[
            pl.BlockSpec((BLOCK, 1), lambda i: (i, 0)),
            pl.BlockSpec((BLOCK, 1), lambda i: (i, 0)),
        ],
        out_shape=[
            jax.ShapeDtypeStruct((N_NODES, 1), jnp.float32),
            jax.ShapeDtypeStruct((N_NODES, 1), jnp.float32),
        ],
    )(feats, W, b2)


def _set_idx(idx_buf, cur):
    lanes = lax.iota(jnp.int32, L)
    plsc.store_scatter(idx_buf, [jnp.zeros((L,), jnp.int32)],
                       jnp.full((L,), cur, jnp.int32), mask=lanes == 0)


NJ = IN_FEATS // L


def _flush(accs, acc_buf, idx_buf, hg_acc, cur):
    for j in range(NJ):
        acc_buf[0, pl.ds(j * L, L)] = accs[j]
    _set_idx(idx_buf, cur)
    pltpu.sync_copy(acc_buf, hg_acc.at[idx_buf], add=True)


def _process_chunk(pos, n_groups, carry, feats_flat,
                   row_flat, ids_buf, w_buf, acc_buf, idx_buf, hg_acc, start):
    nrows = n_groups * L
    loc = pos - start
    pltpu.sync_copy(feats_flat.at[pl.ds(pos * IN_FEATS, nrows * IN_FEATS)],
                    row_flat.at[pl.ds(0, nrows * IN_FEATS)])

    def rbody(r, carry):
        cur = carry[0]
        accs = carry[1:]
        s_r = ids_buf[pl.ds(loc + r, L)][0]
        w_r = w_buf[pl.ds(loc + r, L)][0]
        changed = s_r != cur

        @pl.when(changed)
        def _():
            _flush(accs, acc_buf, idx_buf, hg_acc, cur)

        new_accs = tuple(
            jnp.where(changed,
                      w_r * row_flat[pl.ds(r * IN_FEATS + j * L, L)],
                      accs[j] + w_r * row_flat[pl.ds(r * IN_FEATS + j * L, L)])
            for j in range(NJ))
        return (s_r,) + new_accs

    return lax.fori_loop(0, nrows, rbody, carry, unroll=4)


def _sc_body(feats_flat, ids, wvals, zeros_hbm, hg_part,
             row_flat, ids_buf, w_buf, acc_buf, idx_buf, hg_acc):
    cid = lax.axis_index("c")
    sid = lax.axis_index("s")
    wid = cid * 16 + sid

    @pl.when(sid == 0)
    def _():
        pltpu.sync_copy(zeros_hbm, hg_acc)

    plsc.subcore_barrier()

    start = wid * PER_W
    count = jnp.minimum(PER_W, N_NODES - start)
    nfull = count // CHUNK
    ntail = (count - nfull * CHUNK) // L

    pltpu.sync_copy(ids.at[pl.ds(start, PER_W)], ids_buf.at[pl.ds(0, PER_W)])
    pltpu.sync_copy(wvals.at[pl.ds(start, PER_W)], w_buf.at[pl.ds(0, PER_W)])

    args = (feats_flat, row_flat, ids_buf, w_buf, acc_buf, idx_buf, hg_acc,
            start)

    zero = jnp.zeros((L,), jnp.float32)
    carry0 = (jnp.int32(NUM_GRAPHS),) + (zero,) * NJ

    def full_body(k, carry):
        return _process_chunk(start + k * CHUNK, CHUNK // L, carry, *args)

    carry = lax.fori_loop(0, nfull, full_body, carry0)

    def tail_body(k, carry):
        return _process_chunk(start + nfull * CHUNK + k * L, 1, carry, *args)

    carry = lax.fori_loop(0, ntail, tail_body, carry)

    _flush(carry[1:], acc_buf, idx_buf, hg_acc, carry[0])

    plsc.subcore_barrier()
    off = cid * NUM_GRAPHS + sid * L
    pltpu.sync_copy(hg_acc.at[pl.ds(sid * L, L)], hg_part.at[pl.ds(off, L)])


@jax.jit
def _sc_call(feats_flat, ids_pad, w_pad, zeros):
    mesh = plsc.VectorSubcoreMesh(core_axis_name="c", subcore_axis_name="s",
                                  num_cores=2, num_subcores=16)
    return pl.kernel(
        _sc_body,
        out_type=jax.ShapeDtypeStruct((2 * NUM_GRAPHS, IN_FEATS),
                                      jnp.float32),
        mesh=mesh,
        compiler_params=pltpu.CompilerParams(use_tc_tiling_on_sc=False,
                                             needs_layout_passes=False),
        scratch_types=[
            pltpu.VMEM((CHUNK * IN_FEATS,), jnp.float32),
            pltpu.VMEM((PER_W + L,), jnp.int32),
            pltpu.VMEM((PER_W + L,), jnp.float32),
            pltpu.VMEM((1, IN_FEATS), jnp.float32),
            pltpu.VMEM((1,), jnp.int32),
            pltpu.VMEM_SHARED((NUM_GRAPHS + 1, IN_FEATS), jnp.float32),
        ],
    )(feats_flat, ids_pad, w_pad, zeros)


def _merge_body(p_ref, out_ref):
    out_ref[...] = p_ref[0] + p_ref[1]


def kernel(feats, segment_ids, W, b):
    b2 = b.reshape(1, 1).astype(jnp.float32)
    aw, wv = _tc_matvec(feats, W, b2)
    ids_pad = jnp.pad(segment_ids.astype(jnp.int32), (0, N_PAD - N_NODES),
                      constant_values=NUM_GRAPHS - 1)
    w_pad = jnp.pad(wv.reshape(N_NODES), (0, N_PAD - N_NODES))
    zeros = jnp.zeros((NUM_GRAPHS + 1, IN_FEATS), jnp.float32)
    hg_part = _sc_call(feats.reshape(N_NODES * IN_FEATS), ids_pad, w_pad,
                       zeros)
    hg = pl.pallas_call(
        _merge_body,
        out_shape=jax.ShapeDtypeStruct((NUM_GRAPHS, IN_FEATS), jnp.float32),
    )(hg_part.reshape(2, NUM_GRAPHS, IN_FEATS))
    return (hg, aw)


# speedup vs baseline: 3.1439x; 1.0008x over previous
import jax
import jax.numpy as jnp
from jax import lax
from jax.experimental import pallas as pl
from jax.experimental.pallas import tpu as pltpu
from jax.experimental.pallas import tpu_sc as plsc

N_NODES = 50000
IN_FEATS = 512
NUM_GRAPHS = 256
NW = 32
PER_W = 1568
N_PAD = NW * PER_W
CHUNK = 192
L = 16
BLOCK = 2000
NUM_BLOCKS = N_NODES // BLOCK


def _matvec_body(f_ref, w_ref, b_ref, aw_ref, sig_ref):
    aw = jax.lax.dot_general(
        f_ref[...], w_ref[...], (((1,), (0,)), ((), ())),
        preferred_element_type=jnp.float32,
    ) + b_ref[0, 0]
    aw_ref[...] = aw
    sig_ref[...] = jax.nn.sigmoid(aw)


def _tc_matvec(feats, W, b2):
    return pl.pallas_call(
        _matvec_body,
        grid=(NUM_BLOCKS,),
        in_specs=[
            pl.BlockSpec((BLOCK, IN_FEATS), lambda i: (i, 0)),
            pl.BlockSpec((IN_FEATS, 1), lambda i: (0, 0)),
            pl.BlockSpec((1, 1), lambda i: (0, 0)),
        ],
        out_specs=[
            pl.BlockSpec((BLOCK, 1), lambda i: (i, 0)),
            pl.BlockSpec((BLOCK, 1), lambda i: (i, 0)),
        ],
        out_shape=[
            jax.ShapeDtypeStruct((N_NODES, 1), jnp.float32),
            jax.ShapeDtypeStruct((N_NODES, 1), jnp.float32),
        ],
    )(feats, W, b2)


def _set_idx(idx_buf, cur):
    lanes = lax.iota(jnp.int32, L)
    plsc.store_scatter(idx_buf, [jnp.zeros((L,), jnp.int32)],
                       jnp.full((L,), cur, jnp.int32), mask=lanes == 0)


NJ = IN_FEATS // L


def _flush(accs, acc_buf, idx_buf, hg_acc, cur):
    for j in range(NJ):
        acc_buf[0, pl.ds(j * L, L)] = accs[j]
    _set_idx(idx_buf, cur)
    pltpu.sync_copy(acc_buf, hg_acc.at[idx_buf], add=True)


def _process_chunk(pos, n_groups, carry, feats2d,
                   row_buf, ids_buf, w_buf, acc_buf, idx_buf, hg_acc, start):
    nrows = n_groups * L
    loc = pos - start
    pltpu.sync_copy(feats2d.at[pl.ds(pos, nrows)],
                    row_buf.at[pl.ds(0, nrows)])

    def rbody(r, carry):
        cur = carry[0]
        accs = carry[1:]
        s_r = ids_buf[pl.ds(loc + r, L)][0]
        w_r = w_buf[pl.ds(loc + r, L)][0]
        changed = s_r != cur

        @pl.when(changed)
        def _():
            _flush(accs, acc_buf, idx_buf, hg_acc, cur)

        new_accs = tuple(
            jnp.where(changed,
                      w_r * row_buf[r, pl.ds(j * L, L)],
                      accs[j] + w_r * row_buf[r, pl.ds(j * L, L)])
            for j in range(NJ))
        return (s_r,) + new_accs

    return lax.fori_loop(0, nrows, rbody, carry, unroll=4)


def _sc_body(feats2d, ids, wvals, zeros_hbm, hg_part,
             row_buf, ids_buf, w_buf, acc_buf, idx_buf, hg_acc):
    cid = lax.axis_index("c")
    sid = lax.axis_index("s")
    wid = cid * 16 + sid

    @pl.when(sid == 0)
    def _():
        pltpu.sync_copy(zeros_hbm, hg_acc)

    plsc.subcore_barrier()

    start = wid * PER_W
    count = jnp.minimum(PER_W, N_NODES - start)
    nfull = count // CHUNK
    ntail = (count - nfull * CHUNK) // L

    pltpu.sync_copy(ids.at[pl.ds(start, PER_W)], ids_buf.at[pl.ds(0, PER_W)])
    pltpu.sync_copy(wvals.at[pl.ds(start, PER_W)], w_buf.at[pl.ds(0, PER_W)])

    args = (feats2d, row_buf, ids_buf, w_buf, acc_buf, idx_buf, hg_acc,
            start)

    zero = jnp.zeros((L,), jnp.float32)
    carry0 = (jnp.int32(NUM_GRAPHS),) + (zero,) * NJ

    def full_body(k, carry):
        return _process_chunk(start + k * CHUNK, CHUNK // L, carry, *args)

    carry = lax.fori_loop(0, nfull, full_body, carry0)

    def tail_body(k, carry):
        return _process_chunk(start + nfull * CHUNK + k * L, 1, carry, *args)

    carry = lax.fori_loop(0, ntail, tail_body, carry)

    _flush(carry[1:], acc_buf, idx_buf, hg_acc, carry[0])

    plsc.subcore_barrier()
    off = cid * NUM_GRAPHS + sid * L
    pltpu.sync_copy(hg_acc.at[pl.ds(sid * L, L)], hg_part.at[pl.ds(off, L)])


@jax.jit
def _sc_call(feats2d, ids_pad, w_pad, zeros):
    mesh = plsc.VectorSubcoreMesh(core_axis_name="c", subcore_axis_name="s",
                                  num_cores=2, num_subcores=16)
    return pl.kernel(
        _sc_body,
        out_type=jax.ShapeDtypeStruct((2 * NUM_GRAPHS, IN_FEATS),
                                      jnp.float32),
        mesh=mesh,
        compiler_params=pltpu.CompilerParams(use_tc_tiling_on_sc=False,
                                             needs_layout_passes=False),
        scratch_types=[
            pltpu.VMEM((CHUNK, IN_FEATS), jnp.float32),
            pltpu.VMEM((PER_W + L,), jnp.int32),
            pltpu.VMEM((PER_W + L,), jnp.float32),
            pltpu.VMEM((1, IN_FEATS), jnp.float32),
            pltpu.VMEM((1,), jnp.int32),
            pltpu.VMEM_SHARED((NUM_GRAPHS + 1, IN_FEATS), jnp.float32),
        ],
    )(feats2d, ids_pad, w_pad, zeros)


def _merge_body(p_ref, out_ref):
    out_ref[...] = p_ref[0] + p_ref[1]


def kernel(feats, segment_ids, W, b):
    b2 = b.reshape(1, 1).astype(jnp.float32)
    aw, wv = _tc_matvec(feats, W, b2)
    ids_pad = jnp.pad(segment_ids.astype(jnp.int32), (0, N_PAD - N_NODES),
                      constant_values=NUM_GRAPHS - 1)
    w_pad = jnp.pad(wv.reshape(N_NODES), (0, N_PAD - N_NODES))
    zeros = jnp.zeros((NUM_GRAPHS + 1, IN_FEATS), jnp.float32)
    hg_part = _sc_call(feats, ids_pad, w_pad, zeros)
    hg = pl.pallas_call(
        _merge_body,
        out_shape=jax.ShapeDtypeStruct((NUM_GRAPHS, IN_FEATS), jnp.float32),
    )(hg_part.reshape(2, NUM_GRAPHS, IN_FEATS))
    return (hg, aw)


# speedup vs baseline: 3.3974x; 1.0806x over previous
import jax
import jax.numpy as jnp
from jax import lax
from jax.experimental import pallas as pl
from jax.experimental.pallas import tpu as pltpu
from jax.experimental.pallas import tpu_sc as plsc

N_NODES = 50000
IN_FEATS = 512
NUM_GRAPHS = 256
NW = 32
PER_W = 1568
N_PAD = NW * PER_W
CHUNK = 192
L = 16
NJ = IN_FEATS // L
BLOCK = 2000
NUM_BLOCKS = N_NODES // BLOCK
ZROWS = 16


def _matvec_body(f_ref, w_ref, b_ref, aw_ref, sig_ref):
    aw = jax.lax.dot_general(
        f_ref[...], w_ref[...], (((1,), (0,)), ((), ())),
        preferred_element_type=jnp.float32,
    ) + b_ref[0, 0]
    aw_ref[...] = aw
    sig_ref[...] = jax.nn.sigmoid(aw)


def _tc_matvec(feats, W, b2):
    return pl.pallas_call(
        _matvec_body,
        grid=(NUM_BLOCKS,),
        in_specs=[
            pl.BlockSpec((BLOCK, IN_FEATS), lambda i: (i, 0)),
            pl.BlockSpec((IN_FEATS, 1), lambda i: (0, 0)),
            pl.BlockSpec((1, 1), lambda i: (0, 0)),
        ],
        out_specs=[
            pl.BlockSpec((BLOCK, 1), lambda i: (i, 0)),
            pl.BlockSpec((BLOCK, 1), lambda i: (i, 0)),
        ],
        out_shape=[
            jax.ShapeDtypeStruct((N_NODES, 1), jnp.float32),
            jax.ShapeDtypeStruct((N_NODES, 1), jnp.float32),
        ],
    )(feats, W, b2)


def _flush(accs, acc_buf, parts, region, cur):
    for j in range(NJ):
        acc_buf[pl.ds(j * L, L)] = accs[j]
    pltpu.sync_copy(acc_buf, parts.at[pl.ds(region + cur * IN_FEATS,
                                            IN_FEATS)])


def _process_chunk(pos, n_groups, carry, feats2d,
                   row_buf, ids_buf, w_buf, acc_buf, parts, region, start):
    nrows = n_groups * L
    loc = pos - start
    pltpu.sync_copy(feats2d.at[pl.ds(pos, nrows)],
                    row_buf.at[pl.ds(0, nrows)])

    def rbody(r, carry):
        cur = carry[0]
        accs = carry[1:]
        s_r = ids_buf[pl.ds(loc + r, L)][0]
        w_r = w_buf[pl.ds(loc + r, L)][0]
        changed = s_r != cur

        @pl.when(changed)
        def _():
            _flush(accs, acc_buf, parts, region, cur)

        new_accs = tuple(
            jnp.where(changed,
                      w_r * row_buf[r, pl.ds(j * L, L)],
                      accs[j] + w_r * row_buf[r, pl.ds(j * L, L)])
            for j in range(NJ))
        return (s_r,) + new_accs

    return lax.fori_loop(0, nrows, rbody, carry, unroll=4)


def _sc_body(feats2d, ids, wvals, parts,
             row_buf, ids_buf, w_buf, acc_buf, zero_buf):
    cid = lax.axis_index("c")
    sid = lax.axis_index("s")
    wid = cid * 16 + sid
    region = wid * NUM_GRAPHS * IN_FEATS

    z = jnp.zeros((L,), jnp.float32)
    for j in range(ZROWS * NJ):
        zero_buf[pl.ds(j * L, L)] = z
    for k in range(NUM_GRAPHS // ZROWS):
        pltpu.sync_copy(
            zero_buf,
            parts.at[pl.ds(region + k * ZROWS * IN_FEATS, ZROWS * IN_FEATS)])

    start = wid * PER_W
    count = jnp.minimum(PER_W, N_NODES - start)
    nfull = count // CHUNK
    ntail = (count - nfull * CHUNK) // L

    pltpu.sync_copy(ids.at[pl.ds(start, PER_W)], ids_buf.at[pl.ds(0, PER_W)])
    pltpu.sync_copy(wvals.at[pl.ds(start, PER_W)], w_buf.at[pl.ds(0, PER_W)])

    args = (feats2d, row_buf, ids_buf, w_buf, acc_buf, parts, region, start)

    zero = jnp.zeros((L,), jnp.float32)
    carry0 = (jnp.int32(0),) + (zero,) * NJ

    def full_body(k, carry):
        return _process_chunk(start + k * CHUNK, CHUNK // L, carry, *args)

    carry = lax.fori_loop(0, nfull, full_body, carry0)

    def tail_body(k, carry):
        return _process_chunk(start + nfull * CHUNK + k * L, 1, carry, *args)

    carry = lax.fori_loop(0, ntail, tail_body, carry)

    _flush(carry[1:], acc_buf, parts, region, carry[0])


@jax.jit
def _sc_call(feats2d, ids_pad, w_pad):
    mesh = plsc.VectorSubcoreMesh(core_axis_name="c", subcore_axis_name="s",
                                  num_cores=2, num_subcores=16)
    return pl.kernel(
        _sc_body,
        out_type=jax.ShapeDtypeStruct((NW * NUM_GRAPHS * IN_FEATS,),
                                      jnp.float32),
        mesh=mesh,
        compiler_params=pltpu.CompilerParams(use_tc_tiling_on_sc=True,
                                             needs_layout_passes=False),
        scratch_types=[
            pltpu.VMEM((CHUNK, IN_FEATS), jnp.float32),
            pltpu.VMEM((PER_W + L,), jnp.int32),
            pltpu.VMEM((PER_W + L,), jnp.float32),
            pltpu.VMEM((IN_FEATS,), jnp.float32),
            pltpu.VMEM((ZROWS * IN_FEATS,), jnp.float32),
        ],
    )(feats2d, ids_pad, w_pad)


def _merge_body(p_ref, out_ref):
    i = pl.program_id(0)

    @pl.when(i == 0)
    def _():
        out_ref[...] = p_ref[0]

    @pl.when(i > 0)
    def _():
        out_ref[...] += p_ref[0]


def kernel(feats, segment_ids, W, b):
    b2 = b.reshape(1, 1).astype(jnp.float32)
    aw, wv = _tc_matvec(feats, W, b2)
    ids_pad = jnp.pad(segment_ids.astype(jnp.int32), (0, N_PAD - N_NODES),
                      constant_values=NUM_GRAPHS - 1)
    w_pad = jnp.pad(wv.reshape(N_NODES), (0, N_PAD - N_NODES))
    parts = _sc_call(feats, ids_pad, w_pad)
    hg = pl.pallas_call(
        _merge_body,
        grid=(NW,),
        in_specs=[pl.BlockSpec((1, NUM_GRAPHS, IN_FEATS), lambda i: (i, 0, 0))],
        out_specs=pl.BlockSpec((NUM_GRAPHS, IN_FEATS), lambda i: (0, 0)),
        out_shape=jax.ShapeDtypeStruct((NUM_GRAPHS, IN_FEATS), jnp.float32),
    )(parts.reshape(NW, NUM_GRAPHS, IN_FEATS))
    return (hg, aw)


# speedup vs baseline: 4.0005x; 1.1775x over previous
import jax
import jax.numpy as jnp
from jax import lax
from jax.experimental import pallas as pl
from jax.experimental.pallas import tpu as pltpu
from jax.experimental.pallas import tpu_sc as plsc

N_NODES = 50000
IN_FEATS = 512
NUM_GRAPHS = 256
NW = 32
PER_W = 1568
N_PAD = NW * PER_W
CHUNK = 96
L = 16
NJ = IN_FEATS // L
BLOCK = 2000
NUM_BLOCKS = N_NODES // BLOCK
ZROWS = 16


def _matvec_body(f_ref, w_ref, b_ref, aw_ref, sig_ref):
    aw = jax.lax.dot_general(
        f_ref[...], w_ref[...], (((1,), (0,)), ((), ())),
        preferred_element_type=jnp.float32,
    ) + b_ref[0, 0]
    aw_ref[...] = aw
    sig_ref[...] = jax.nn.sigmoid(aw)


def _tc_matvec(feats, W, b2):
    return pl.pallas_call(
        _matvec_body,
        grid=(NUM_BLOCKS,),
        in_specs=[
            pl.BlockSpec((BLOCK, IN_FEATS), lambda i: (i, 0)),
            pl.BlockSpec((IN_FEATS, 1), lambda i: (0, 0)),
            pl.BlockSpec((1, 1), lambda i: (0, 0)),
        ],
        out_specs=[
            pl.BlockSpec((BLOCK, 1), lambda i: (i, 0)),
            pl.BlockSpec((BLOCK, 1), lambda i: (i, 0)),
        ],
        out_shape=[
            jax.ShapeDtypeStruct((N_NODES, 1), jnp.float32),
            jax.ShapeDtypeStruct((N_NODES, 1), jnp.float32),
        ],
    )(feats, W, b2)


def _flush(accs, acc_buf, parts, region, cur):
    for j in range(NJ):
        acc_buf[pl.ds(j * L, L)] = accs[j]
    pltpu.sync_copy(acc_buf, parts.at[pl.ds(region + cur * IN_FEATS,
                                            IN_FEATS)])


def _process_rows(n_groups, loc, carry, row_buf, ids_buf, w_buf,
                  acc_buf, parts, region):
    nrows = n_groups * L

    def rbody(r, carry):
        cur = carry[0]
        accs = carry[1:]
        s_r = ids_buf[pl.ds(loc + r, L)][0]
        w_r = w_buf[pl.ds(loc + r, L)][0]
        changed = s_r != cur

        @pl.when(changed)
        def _():
            _flush(accs, acc_buf, parts, region, cur)

        new_accs = tuple(
            jnp.where(changed,
                      w_r * row_buf[r, pl.ds(j * L, L)],
                      accs[j] + w_r * row_buf[r, pl.ds(j * L, L)])
            for j in range(NJ))
        return (s_r,) + new_accs

    return lax.fori_loop(0, nrows, rbody, carry, unroll=4)


def _sc_body(feats2d, ids, wvals, parts,
             row_buf0, row_buf1, ids_buf, w_buf, acc_buf, zero_buf,
             sem0, sem1):
    cid = lax.axis_index("c")
    sid = lax.axis_index("s")
    wid = cid * 16 + sid
    region = wid * NUM_GRAPHS * IN_FEATS

    z = jnp.zeros((L,), jnp.float32)
    for j in range(ZROWS * NJ):
        zero_buf[pl.ds(j * L, L)] = z
    for k in range(NUM_GRAPHS // ZROWS):
        pltpu.sync_copy(
            zero_buf,
            parts.at[pl.ds(region + k * ZROWS * IN_FEATS, ZROWS * IN_FEATS)])

    start = wid * PER_W
    count = jnp.minimum(PER_W, N_NODES - start)
    nfull = count // CHUNK
    ntail = (count - nfull * CHUNK) // L

    pltpu.sync_copy(ids.at[pl.ds(start, PER_W)], ids_buf.at[pl.ds(0, PER_W)])
    pltpu.sync_copy(wvals.at[pl.ds(start, PER_W)], w_buf.at[pl.ds(0, PER_W)])

    zero = jnp.zeros((L,), jnp.float32)
    carry0 = (jnp.int32(0),) + (zero,) * NJ

    def dma_start(buf, sem, k):
        pltpu.async_copy(feats2d.at[pl.ds(start + k * CHUNK, CHUNK)], buf,
                         sem)

    def dma_wait(buf, sem):
        pltpu.make_async_copy(feats2d.at[pl.ds(0, CHUNK)], buf, sem).wait()

    def proc(buf, k, n_groups, carry):
        return _process_rows(n_groups, k * CHUNK, carry, buf, ids_buf,
                             w_buf, acc_buf, parts, region)

    dma_start(row_buf0, sem0, 0)

    def pair_body(k2, carry):
        k = 2 * k2
        dma_wait(row_buf0, sem0)
        dma_start(row_buf1, sem1, k + 1)
        carry = proc(row_buf0, k, CHUNK // L, carry)
        dma_wait(row_buf1, sem1)

        @pl.when(k + 2 < nfull)
        def _():
            dma_start(row_buf0, sem0, k + 2)

        return proc(row_buf1, k + 1, CHUNK // L, carry)

    carry = lax.cond(nfull > 0,
                     lambda c: lax.fori_loop(0, nfull // 2, pair_body, c),
                     lambda c: c, carry0)

    def tail_body(k, carry):
        pos = start + nfull * CHUNK + k * L
        pltpu.sync_copy(feats2d.at[pl.ds(pos, L)], row_buf0.at[pl.ds(0, L)])
        return _process_rows(1, pos - start, carry,
                             row_buf0, ids_buf, w_buf, acc_buf, parts,
                             region)

    carry = lax.fori_loop(0, ntail, tail_body, carry)

    _flush(carry[1:], acc_buf, parts, region, carry[0])


@jax.jit
def _sc_call(feats2d, ids_pad, w_pad):
    mesh = plsc.VectorSubcoreMesh(core_axis_name="c", subcore_axis_name="s",
                                  num_cores=2, num_subcores=16)
    return pl.kernel(
        _sc_body,
        out_type=jax.ShapeDtypeStruct((NW * NUM_GRAPHS * IN_FEATS,),
                                      jnp.float32),
        mesh=mesh,
        compiler_params=pltpu.CompilerParams(use_tc_tiling_on_sc=True,
                                             needs_layout_passes=False),
        scratch_types=[
            pltpu.VMEM((CHUNK, IN_FEATS), jnp.float32),
            pltpu.VMEM((CHUNK, IN_FEATS), jnp.float32),
            pltpu.VMEM((PER_W + L,), jnp.int32),
            pltpu.VMEM((PER_W + L,), jnp.float32),
            pltpu.VMEM((IN_FEATS,), jnp.float32),
            pltpu.VMEM((ZROWS * IN_FEATS,), jnp.float32),
            pltpu.SemaphoreType.DMA,
            pltpu.SemaphoreType.DMA,
        ],
    )(feats2d, ids_pad, w_pad)


def _merge_body(p_ref, out_ref):
    i = pl.program_id(0)

    @pl.when(i == 0)
    def _():
        out_ref[...] = p_ref[0]

    @pl.when(i > 0)
    def _():
        out_ref[...] += p_ref[0]


def kernel(feats, segment_ids, W, b):
    b2 = b.reshape(1, 1).astype(jnp.float32)
    aw, wv = _tc_matvec(feats, W, b2)
    ids_pad = jnp.pad(segment_ids.astype(jnp.int32), (0, N_PAD - N_NODES),
                      constant_values=NUM_GRAPHS - 1)
    w_pad = jnp.pad(wv.reshape(N_NODES), (0, N_PAD - N_NODES))
    parts = _sc_call(feats, ids_pad, w_pad)
    hg = pl.pallas_call(
        _merge_body,
        grid=(NW,),
        in_specs=[pl.BlockSpec((1, NUM_GRAPHS, IN_FEATS), lambda i: (i, 0, 0))],
        out_specs=pl.BlockSpec((NUM_GRAPHS, IN_FEATS), lambda i: (0, 0)),
        out_shape=jax.ShapeDtypeStruct((NUM_GRAPHS, IN_FEATS), jnp.float32),
    )(parts.reshape(NW, NUM_GRAPHS, IN_FEATS))
    return (hg, aw)
